# baseline (speedup 1.0000x reference)
import functools

import jax
import jax.numpy as jnp
from jax import lax
from jax.experimental import pallas as pl
from jax.experimental.pallas import tpu as pltpu
from jax.experimental.pallas import tpu_sc as plsc

N = 10000
E = 160000
D = 512
DS = 128
NSLAB = D // DS
NC, NS = 2, 16
CHUNK = 128
NCHP = 1280
E_PAD = NCHP * CHUNK
NP = N + 8
AGG_NLOC = NCHP // NS
DEG_NLOC = NCHP // (NC * NS)
ZROWS = N // NS
BM = 1000
NBUF = 2
AGG_HALF = AGG_NLOC // 2
DEG_RING = 8


def _sc_degree(dst2d, ones_src, zeros_src):
    mesh = plsc.VectorSubcoreMesh(core_axis_name="c", subcore_axis_name="s")

    @functools.partial(
        pl.kernel,
        mesh=mesh,
        out_type=jax.ShapeDtypeStruct((NC, NS, ZROWS, DS), jnp.float32),
        scratch_types=[
            pltpu.VMEM((DEG_NLOC, CHUNK), jnp.int32),
            pltpu.VMEM((CHUNK, DS), jnp.float32),
            pltpu.VMEM_SHARED((NP, DS), jnp.float32),
        ]
        + [pltpu.SemaphoreType.DMA] * DEG_RING,
    )
    def k(dst_hbm, ones_hbm, zeros_hbm, out_hbm, di_all, ones_v, acc_sh, *sems):
        c = lax.axis_index("c")
        s = lax.axis_index("s")
        pltpu.sync_copy(ones_hbm, ones_v)
        pltpu.sync_copy(
            dst_hbm.at[pl.ds((c * NS + s) * DEG_NLOC, DEG_NLOC)], di_all)
        pltpu.sync_copy(zeros_hbm, acc_sh.at[pl.ds(s * ZROWS, ZROWS)])
        plsc.subcore_barrier()

        @pl.loop(0, DEG_NLOC // DEG_RING)
        def _(t):
            hs = []
            for bq in range(DEG_RING):
                hs.append(pltpu.async_copy(
                    ones_v, acc_sh.at[di_all.at[t * DEG_RING + bq]],
                    sems[bq], add=True))
            for bq in range(DEG_RING):
                hs[bq].wait()

        plsc.subcore_barrier()
        pltpu.sync_copy(acc_sh.at[pl.ds(s * ZROWS, ZROWS)], out_hbm.at[c, s])

    return k(dst2d, ones_src, zeros_src)


def _sc_aggregate(g4, src2d, dst2d, zeros_src):
    mesh = plsc.VectorSubcoreMesh(core_axis_name="c", subcore_axis_name="s")

    @functools.partial(
        pl.kernel,
        mesh=mesh,
        out_type=jax.ShapeDtypeStruct((NSLAB, NS, ZROWS, DS), jnp.float32),
        scratch_types=[
            pltpu.VMEM((AGG_HALF, CHUNK), jnp.int32),
            pltpu.VMEM((AGG_HALF, CHUNK), jnp.int32),
        ]
        + [pltpu.VMEM((CHUNK, DS), jnp.float32)] * NBUF
        + [pltpu.VMEM_SHARED((NP, DS), jnp.float32)]
        + [pltpu.SemaphoreType.DMA] * (2 * NBUF),
    )
    def k(g_hbm, src_hbm, dst_hbm, z_hbm, out_hbm, si_h, di_h, *rest):
        rows = rest[:NBUF]
        acc_sh = rest[NBUF]
        gsem = rest[NBUF + 1:NBUF + 1 + NBUF]
        ssem = rest[NBUF + 1 + NBUF:]
        c = lax.axis_index("c")
        s = lax.axis_index("s")
        for p in range(NSLAB // NC):
            slab = c * (NSLAB // NC) + p
            pltpu.sync_copy(z_hbm, acc_sh.at[pl.ds(s * ZROWS, ZROWS)])
            plsc.subcore_barrier()

            for half in range(2):
                base = s * AGG_NLOC + half * AGG_HALF
                pltpu.sync_copy(src_hbm.at[pl.ds(base, AGG_HALF)], si_h)
                pltpu.sync_copy(dst_hbm.at[pl.ds(base, AGG_HALF)], di_h)

                @pl.loop(0, AGG_HALF // NBUF)
                def _(t):
                    gh = []
                    for bq in range(NBUF):
                        gh.append(pltpu.async_copy(
                            g_hbm.at[slab].at[si_h.at[t * NBUF + bq]],
                            rows[bq], gsem[bq]))
                    sh = []
                    for bq in range(NBUF):
                        gh[bq].wait()
                        sh.append(pltpu.async_copy(
                            rows[bq], acc_sh.at[di_h.at[t * NBUF + bq]],
                            ssem[bq], add=True))
                    for bq in range(NBUF):
                        sh[bq].wait()

            plsc.subcore_barrier()
            pltpu.sync_copy(acc_sh.at[pl.ds(s * ZROWS, ZROWS)], out_hbm.at[slab, s])
            plsc.subcore_barrier()

    return k(g4, src2d, dst2d, zeros_src)


def _tc_transform(x, W, degp):
    def body(x_ref, w_ref, da_ref, db_ref, g_ref, slh_ref):
        h = jnp.dot(x_ref[...], w_ref[...], preferred_element_type=jnp.float32)
        deg = 1.0 + da_ref[0, :, 0] + db_ref[0, :, 0]
        dis = lax.rsqrt(deg)[:, None]
        g = h * dis
        slh_ref[...] = g * dis
        for p in range(NSLAB):
            g_ref[p, :, :] = g[:, p * DS:(p + 1) * DS]

    return pl.pallas_call(
        body,
        grid=(N // BM,),
        in_specs=[
            pl.BlockSpec((BM, D), lambda i: (i, 0)),
            pl.BlockSpec((D, D), lambda i: (0, 0)),
            pl.BlockSpec((1, BM, DS), lambda i: (0, i, 0)),
            pl.BlockSpec((1, BM, DS), lambda i: (1, i, 0)),
        ],
        out_specs=[
            pl.BlockSpec((NSLAB, BM, DS), lambda i: (0, i, 0)),
            pl.BlockSpec((BM, D), lambda i: (i, 0)),
        ],
        out_shape=[
            jax.ShapeDtypeStruct((NSLAB, N, DS), jnp.float32),
            jax.ShapeDtypeStruct((N, D), jnp.float32),
        ],
    )(x, W, degp, degp)


def _tc_combine(acc4, slh, degp, b_row):
    def body(a_ref, slh_ref, da_ref, db_ref, b_ref, o_ref):
        deg = 1.0 + da_ref[0, :, 0] + db_ref[0, :, 0]
        dis = lax.rsqrt(deg)[:, None]
        acc = jnp.concatenate([a_ref[p] for p in range(NSLAB)], axis=1)
        o_ref[...] = acc * dis + slh_ref[...] + b_ref[...]

    return pl.pallas_call(
        body,
        grid=(N // BM,),
        in_specs=[
            pl.BlockSpec((NSLAB, BM, DS), lambda i: (0, i, 0)),
            pl.BlockSpec((BM, D), lambda i: (i, 0)),
            pl.BlockSpec((1, BM, DS), lambda i: (0, i, 0)),
            pl.BlockSpec((1, BM, DS), lambda i: (1, i, 0)),
            pl.BlockSpec((1, D), lambda i: (0, 0)),
        ],
        out_specs=pl.BlockSpec((BM, D), lambda i: (i, 0)),
        out_shape=jax.ShapeDtypeStruct((N, D), jnp.float32),
    )(acc4, slh, degp, degp, b_row)


def kernel(x, edge_index, W, b):
    src = edge_index[0].astype(jnp.int32)
    dst = edge_index[1].astype(jnp.int32)
    src2 = jnp.concatenate(
        [src, jnp.zeros((E_PAD - E,), jnp.int32)]).reshape(NCHP, CHUNK)
    dst2 = jnp.concatenate(
        [dst, jnp.full((E_PAD - E,), N, jnp.int32)]).reshape(NCHP, CHUNK)
    ones_src = jnp.ones((CHUNK, DS), jnp.float32)
    zeros_rows = jnp.zeros((ZROWS, DS), jnp.float32)

    degp = _sc_degree(dst2, ones_src, zeros_rows).reshape(NC, N, DS)
    g4, slh = _tc_transform(x, W, degp)
    acc4 = _sc_aggregate(g4, src2, dst2, zeros_rows).reshape(NSLAB, N, DS)
    out = _tc_combine(acc4, slh, degp, b[None, :])
    return out


# speedup vs baseline: 1.1083x; 1.1083x over previous
import functools

import jax
import jax.numpy as jnp
from jax import lax
from jax.experimental import pallas as pl
from jax.experimental.pallas import tpu as pltpu
from jax.experimental.pallas import tpu_sc as plsc

N = 10000
E = 160000
D = 512
DS = 128
NSLAB = D // DS
NC, NS = 2, 16
CHUNK = 128
NCHP = 1280
E_PAD = NCHP * CHUNK
NP = N + 8
AGG_NLOC = NCHP // NS
DEG_NLOC = NCHP // (NC * NS)
ZROWS = N // NS
BM = 1000
NBUF = 2
AGG_HALF = AGG_NLOC // 2
DEG_RING = 8


def _sc_degree(dst2d, ones_src, zeros_src):
    mesh = plsc.VectorSubcoreMesh(core_axis_name="c", subcore_axis_name="s")

    @functools.partial(
        pl.kernel,
        mesh=mesh,
        out_type=jax.ShapeDtypeStruct((NC, NS, ZROWS, DS), jnp.float32),
        scratch_types=[
            pltpu.VMEM((DEG_NLOC, CHUNK), jnp.int32),
            pltpu.VMEM((CHUNK, DS), jnp.float32),
            pltpu.VMEM_SHARED((NP, DS), jnp.float32),
        ]
        + [pltpu.SemaphoreType.DMA] * DEG_RING,
    )
    def k(dst_hbm, ones_hbm, zeros_hbm, out_hbm, di_all, ones_v, acc_sh, *sems):
        c = lax.axis_index("c")
        s = lax.axis_index("s")
        pltpu.sync_copy(ones_hbm, ones_v)
        pltpu.sync_copy(
            dst_hbm.at[pl.ds((c * NS + s) * DEG_NLOC, DEG_NLOC)], di_all)
        pltpu.sync_copy(zeros_hbm, acc_sh.at[pl.ds(s * ZROWS, ZROWS)])
        plsc.subcore_barrier()

        @pl.loop(0, DEG_NLOC // DEG_RING)
        def _(t):
            hs = []
            for bq in range(DEG_RING):
                hs.append(pltpu.async_copy(
                    ones_v, acc_sh.at[di_all.at[t * DEG_RING + bq]],
                    sems[bq], add=True))
            for bq in range(DEG_RING):
                hs[bq].wait()

        plsc.subcore_barrier()
        pltpu.sync_copy(acc_sh.at[pl.ds(s * ZROWS, ZROWS)], out_hbm.at[c, s])

    return k(dst2d, ones_src, zeros_src)


def _sc_aggregate(g4, src2d, dst2d, zeros_src):
    mesh = plsc.VectorSubcoreMesh(core_axis_name="c", subcore_axis_name="s")

    @functools.partial(
        pl.kernel,
        mesh=mesh,
        out_type=jax.ShapeDtypeStruct((NSLAB, NS, ZROWS, DS), jnp.float32),
        scratch_types=[
            pltpu.VMEM((AGG_HALF, CHUNK), jnp.int32),
            pltpu.VMEM((AGG_HALF, CHUNK), jnp.int32),
        ]
        + [pltpu.VMEM((CHUNK, DS), jnp.float32)] * NBUF
        + [pltpu.VMEM_SHARED((NP, DS), jnp.float32)]
        + [pltpu.SemaphoreType.DMA] * (2 * NBUF),
    )
    def k(g_hbm, src_hbm, dst_hbm, z_hbm, out_hbm, si_h, di_h, *rest):
        rows = rest[:NBUF]
        acc_sh = rest[NBUF]
        gsem = rest[NBUF + 1:NBUF + 1 + NBUF]
        ssem = rest[NBUF + 1 + NBUF:]
        c = lax.axis_index("c")
        s = lax.axis_index("s")
        for p in range(NSLAB // NC):
            slab = c * (NSLAB // NC) + p
            pltpu.sync_copy(z_hbm, acc_sh.at[pl.ds(s * ZROWS, ZROWS)])
            plsc.subcore_barrier()

            for half in range(2):
                base = s * AGG_NLOC + half * AGG_HALF
                pltpu.sync_copy(src_hbm.at[pl.ds(base, AGG_HALF)], si_h)
                pltpu.sync_copy(dst_hbm.at[pl.ds(base, AGG_HALF)], di_h)
                pltpu.make_async_copy(
                    g_hbm.at[slab].at[si_h.at[0]], rows[0], gsem[0]).start()

                @pl.loop(0, AGG_HALF // 2)
                def _(u):
                    j0 = 2 * u
                    pltpu.make_async_copy(
                        g_hbm.at[slab].at[si_h.at[j0 + 1]], rows[1],
                        gsem[1]).start()
                    pltpu.make_async_copy(
                        g_hbm.at[slab].at[si_h.at[j0]], rows[0],
                        gsem[0]).wait()
                    pltpu.sync_copy(rows[0], acc_sh.at[di_h.at[j0]], add=True)

                    @pl.when(j0 + 2 < AGG_HALF)
                    def _():
                        pltpu.make_async_copy(
                            g_hbm.at[slab].at[si_h.at[j0 + 2]], rows[0],
                            gsem[0]).start()

                    pltpu.make_async_copy(
                        g_hbm.at[slab].at[si_h.at[j0 + 1]], rows[1],
                        gsem[1]).wait()
                    pltpu.sync_copy(
                        rows[1], acc_sh.at[di_h.at[j0 + 1]], add=True)

            plsc.subcore_barrier()
            pltpu.sync_copy(acc_sh.at[pl.ds(s * ZROWS, ZROWS)], out_hbm.at[slab, s])
            plsc.subcore_barrier()

    return k(g4, src2d, dst2d, zeros_src)


def _tc_transform(x, W, degp):
    def body(x_ref, w_ref, da_ref, db_ref, g_ref, slh_ref):
        h = jnp.dot(x_ref[...], w_ref[...], preferred_element_type=jnp.float32)
        deg = 1.0 + da_ref[0, :, 0] + db_ref[0, :, 0]
        dis = lax.rsqrt(deg)[:, None]
        g = h * dis
        slh_ref[...] = g * dis
        for p in range(NSLAB):
            g_ref[p, :, :] = g[:, p * DS:(p + 1) * DS]

    return pl.pallas_call(
        body,
        grid=(N // BM,),
        in_specs=[
            pl.BlockSpec((BM, D), lambda i: (i, 0)),
            pl.BlockSpec((D, D), lambda i: (0, 0)),
            pl.BlockSpec((1, BM, DS), lambda i: (0, i, 0)),
            pl.BlockSpec((1, BM, DS), lambda i: (1, i, 0)),
        ],
        out_specs=[
            pl.BlockSpec((NSLAB, BM, DS), lambda i: (0, i, 0)),
            pl.BlockSpec((BM, D), lambda i: (i, 0)),
        ],
        out_shape=[
            jax.ShapeDtypeStruct((NSLAB, N, DS), jnp.float32),
            jax.ShapeDtypeStruct((N, D), jnp.float32),
        ],
    )(x, W, degp, degp)


def _tc_combine(acc4, slh, degp, b_row):
    def body(a_ref, slh_ref, da_ref, db_ref, b_ref, o_ref):
        deg = 1.0 + da_ref[0, :, 0] + db_ref[0, :, 0]
        dis = lax.rsqrt(deg)[:, None]
        acc = jnp.concatenate([a_ref[p] for p in range(NSLAB)], axis=1)
        o_ref[...] = acc * dis + slh_ref[...] + b_ref[...]

    return pl.pallas_call(
        body,
        grid=(N // BM,),
        in_specs=[
            pl.BlockSpec((NSLAB, BM, DS), lambda i: (0, i, 0)),
            pl.BlockSpec((BM, D), lambda i: (i, 0)),
            pl.BlockSpec((1, BM, DS), lambda i: (0, i, 0)),
            pl.BlockSpec((1, BM, DS), lambda i: (1, i, 0)),
            pl.BlockSpec((1, D), lambda i: (0, 0)),
        ],
        out_specs=pl.BlockSpec((BM, D), lambda i: (i, 0)),
        out_shape=jax.ShapeDtypeStruct((N, D), jnp.float32),
    )(acc4, slh, degp, degp, b_row)


def kernel(x, edge_index, W, b):
    src = edge_index[0].astype(jnp.int32)
    dst = edge_index[1].astype(jnp.int32)
    src2 = jnp.concatenate(
        [src, jnp.zeros((E_PAD - E,), jnp.int32)]).reshape(NCHP, CHUNK)
    dst2 = jnp.concatenate(
        [dst, jnp.full((E_PAD - E,), N, jnp.int32)]).reshape(NCHP, CHUNK)
    ones_src = jnp.ones((CHUNK, DS), jnp.float32)
    zeros_rows = jnp.zeros((ZROWS, DS), jnp.float32)

    degp = _sc_degree(dst2, ones_src, zeros_rows).reshape(NC, N, DS)
    g4, slh = _tc_transform(x, W, degp)
    acc4 = _sc_aggregate(g4, src2, dst2, zeros_rows).reshape(NSLAB, N, DS)
    out = _tc_combine(acc4, slh, degp, b[None, :])
    return out


# speedup vs baseline: 1.1099x; 1.0014x over previous
import functools

import jax
import jax.numpy as jnp
from jax import lax
from jax.experimental import pallas as pl
from jax.experimental.pallas import tpu as pltpu
from jax.experimental.pallas import tpu_sc as plsc

N = 10000
E = 160000
D = 512
DS = 128
NSLAB = D // DS
NC, NS = 2, 16
CHUNK = 128
NCHP = 1280
E_PAD = NCHP * CHUNK
NP = N + 128
AGG_NLOC = NCHP // NS
DEG_NLOC = NCHP // (NC * NS)
ZROWS = N // NS
BM = 1000
NBUF = 2
AGG_HALF = AGG_NLOC // 2
DEG_RING = 8


def _sc_degree(dst2d, ones_src, zeros_src):
    mesh = plsc.VectorSubcoreMesh(core_axis_name="c", subcore_axis_name="s")

    @functools.partial(
        pl.kernel,
        mesh=mesh,
        out_type=jax.ShapeDtypeStruct((NC, NS, ZROWS, DS), jnp.float32),
        scratch_types=[
            pltpu.VMEM((DEG_NLOC, CHUNK), jnp.int32),
            pltpu.VMEM((CHUNK, DS), jnp.float32),
            pltpu.VMEM_SHARED((NP, DS), jnp.float32),
        ]
        + [pltpu.SemaphoreType.DMA] * DEG_RING,
    )
    def k(dst_hbm, ones_hbm, zeros_hbm, out_hbm, di_all, ones_v, acc_sh, *sems):
        c = lax.axis_index("c")
        s = lax.axis_index("s")
        pltpu.sync_copy(ones_hbm, ones_v)
        pltpu.sync_copy(
            dst_hbm.at[pl.ds((c * NS + s) * DEG_NLOC, DEG_NLOC)], di_all)
        pltpu.sync_copy(zeros_hbm, acc_sh.at[pl.ds(s * ZROWS, ZROWS)])
        plsc.subcore_barrier()

        @pl.loop(0, DEG_NLOC // DEG_RING)
        def _(t):
            hs = []
            for bq in range(DEG_RING):
                hs.append(pltpu.async_copy(
                    ones_v, acc_sh.at[di_all.at[t * DEG_RING + bq]],
                    sems[bq], add=True))
            for bq in range(DEG_RING):
                hs[bq].wait()

        plsc.subcore_barrier()
        pltpu.sync_copy(acc_sh.at[pl.ds(s * ZROWS, ZROWS)], out_hbm.at[c, s])

    return k(dst2d, ones_src, zeros_src)


def _sc_aggregate(g4, src2d, dst2d, zeros_src):
    mesh = plsc.VectorSubcoreMesh(core_axis_name="c", subcore_axis_name="s")

    @functools.partial(
        pl.kernel,
        mesh=mesh,
        out_type=jax.ShapeDtypeStruct((NSLAB, NS, ZROWS, DS), jnp.float32),
        scratch_types=[
            pltpu.VMEM((AGG_HALF, CHUNK), jnp.int32),
            pltpu.VMEM((AGG_HALF, CHUNK), jnp.int32),
        ]
        + [pltpu.VMEM((CHUNK, DS), jnp.float32)] * NBUF
        + [pltpu.VMEM_SHARED((NP, DS), jnp.float32)]
        + [pltpu.SemaphoreType.DMA] * (2 * NBUF),
    )
    def k(g_hbm, src_hbm, dst_hbm, z_hbm, out_hbm, si_h, di_h, *rest):
        rows = rest[:NBUF]
        acc_sh = rest[NBUF]
        gsem = rest[NBUF + 1:NBUF + 1 + NBUF]
        ssem = rest[NBUF + 1 + NBUF:]
        c = lax.axis_index("c")
        s = lax.axis_index("s")
        for p in range(NSLAB // NC):
            slab = c * (NSLAB // NC) + p
            pltpu.sync_copy(z_hbm, acc_sh.at[pl.ds(s * ZROWS, ZROWS)])
            plsc.subcore_barrier()

            for half in range(2):
                base = s * AGG_NLOC + half * AGG_HALF
                pltpu.sync_copy(src_hbm.at[pl.ds(base, AGG_HALF)], si_h)
                pltpu.sync_copy(dst_hbm.at[pl.ds(base, AGG_HALF)], di_h)
                pltpu.make_async_copy(
                    g_hbm.at[slab].at[si_h.at[0]], rows[0], gsem[0]).start()

                @pl.loop(0, AGG_HALF // 2)
                def _(u):
                    j0 = 2 * u
                    pltpu.make_async_copy(
                        g_hbm.at[slab].at[si_h.at[j0 + 1]], rows[1],
                        gsem[1]).start()
                    pltpu.make_async_copy(
                        g_hbm.at[slab].at[si_h.at[j0]], rows[0],
                        gsem[0]).wait()
                    pltpu.sync_copy(rows[0], acc_sh.at[di_h.at[j0]], add=True)

                    @pl.when(j0 + 2 < AGG_HALF)
                    def _():
                        pltpu.make_async_copy(
                            g_hbm.at[slab].at[si_h.at[j0 + 2]], rows[0],
                            gsem[0]).start()

                    pltpu.make_async_copy(
                        g_hbm.at[slab].at[si_h.at[j0 + 1]], rows[1],
                        gsem[1]).wait()
                    pltpu.sync_copy(
                        rows[1], acc_sh.at[di_h.at[j0 + 1]], add=True)

            plsc.subcore_barrier()
            pltpu.sync_copy(acc_sh.at[pl.ds(s * ZROWS, ZROWS)], out_hbm.at[slab, s])
            plsc.subcore_barrier()

    return k(g4, src2d, dst2d, zeros_src)


def _tc_transform(x, W, degp):
    def body(x_ref, w_ref, da_ref, db_ref, g_ref, slh_ref):
        h = jnp.dot(x_ref[...], w_ref[...], preferred_element_type=jnp.float32)
        deg = 1.0 + da_ref[0, :, 0] + db_ref[0, :, 0]
        dis = lax.rsqrt(deg)[:, None]
        g = h * dis
        slh_ref[...] = g * dis
        for p in range(NSLAB):
            g_ref[p, :, :] = g[:, p * DS:(p + 1) * DS]

    return pl.pallas_call(
        body,
        grid=(N // BM,),
        in_specs=[
            pl.BlockSpec((BM, D), lambda i: (i, 0)),
            pl.BlockSpec((D, D), lambda i: (0, 0)),
            pl.BlockSpec((1, BM, DS), lambda i: (0, i, 0)),
            pl.BlockSpec((1, BM, DS), lambda i: (1, i, 0)),
        ],
        out_specs=[
            pl.BlockSpec((NSLAB, BM, DS), lambda i: (0, i, 0)),
            pl.BlockSpec((BM, D), lambda i: (i, 0)),
        ],
        out_shape=[
            jax.ShapeDtypeStruct((NSLAB, N, DS), jnp.float32),
            jax.ShapeDtypeStruct((N, D), jnp.float32),
        ],
    )(x, W, degp, degp)


def _tc_combine(acc4, slh, degp, b_row):
    def body(a_ref, slh_ref, da_ref, db_ref, b_ref, o_ref):
        deg = 1.0 + da_ref[0, :, 0] + db_ref[0, :, 0]
        dis = lax.rsqrt(deg)[:, None]
        acc = jnp.concatenate([a_ref[p] for p in range(NSLAB)], axis=1)
        o_ref[...] = acc * dis + slh_ref[...] + b_ref[...]

    return pl.pallas_call(
        body,
        grid=(N // BM,),
        in_specs=[
            pl.BlockSpec((NSLAB, BM, DS), lambda i: (0, i, 0)),
            pl.BlockSpec((BM, D), lambda i: (i, 0)),
            pl.BlockSpec((1, BM, DS), lambda i: (0, i, 0)),
            pl.BlockSpec((1, BM, DS), lambda i: (1, i, 0)),
            pl.BlockSpec((1, D), lambda i: (0, 0)),
        ],
        out_specs=pl.BlockSpec((BM, D), lambda i: (i, 0)),
        out_shape=jax.ShapeDtypeStruct((N, D), jnp.float32),
    )(acc4, slh, degp, degp, b_row)


def kernel(x, edge_index, W, b):
    src = edge_index[0].astype(jnp.int32)
    dst = edge_index[1].astype(jnp.int32)
    src2 = jnp.concatenate(
        [src, jnp.zeros((E_PAD - E,), jnp.int32)]).reshape(NCHP, CHUNK)
    pad_dst = N + (jnp.arange(E_PAD - E, dtype=jnp.int32) % 128)
    dst2 = jnp.concatenate([dst, pad_dst]).reshape(NCHP, CHUNK)
    ones_src = jnp.ones((CHUNK, DS), jnp.float32)
    zeros_rows = jnp.zeros((ZROWS, DS), jnp.float32)

    degp = _sc_degree(dst2, ones_src, zeros_rows).reshape(NC, N, DS)
    g4, slh = _tc_transform(x, W, degp)
    acc4 = _sc_aggregate(g4, src2, dst2, zeros_rows).reshape(NSLAB, N, DS)
    out = _tc_combine(acc4, slh, degp, b[None, :])
    return out


# speedup vs baseline: 1.2804x; 1.1537x over previous
import functools

import jax
import jax.numpy as jnp
from jax import lax
from jax.experimental import pallas as pl
from jax.experimental.pallas import tpu as pltpu
from jax.experimental.pallas import tpu_sc as plsc

N = 10000
E = 160000
D = 512
DS = 128
NSLAB = D // DS
NC, NS = 2, 16
CHUNK = 128
NCHUNKS = E // CHUNK
NCHP = 1280
E_PAD = NCHP * CHUNK
NP = N + 128
AGG_NLOC = NCHP // NS
DEG_NLOC = NCHP // (NC * NS)
ZROWS = N // NS
BM = 1000
NBUF = 2
AGG_HALF = AGG_NLOC // 2
DEG_RING = 8


def _sc_degree(dst2d, ones_src, zeros_src):
    mesh = plsc.VectorSubcoreMesh(core_axis_name="c", subcore_axis_name="s")

    @functools.partial(
        pl.kernel,
        mesh=mesh,
        out_type=jax.ShapeDtypeStruct((NC, NS, ZROWS, DS), jnp.float32),
        scratch_types=[
            pltpu.VMEM((DEG_NLOC, CHUNK), jnp.int32),
            pltpu.VMEM((CHUNK, DS), jnp.float32),
            pltpu.VMEM_SHARED((NP, DS), jnp.float32),
        ]
        + [pltpu.SemaphoreType.DMA] * DEG_RING,
    )
    def k(dst_hbm, ones_hbm, zeros_hbm, out_hbm, di_all, ones_v, acc_sh, *sems):
        c = lax.axis_index("c")
        s = lax.axis_index("s")
        pltpu.sync_copy(ones_hbm, ones_v)
        pltpu.sync_copy(
            dst_hbm.at[pl.ds((c * NS + s) * DEG_NLOC, DEG_NLOC)], di_all)
        pltpu.sync_copy(zeros_hbm, acc_sh.at[pl.ds(s * ZROWS, ZROWS)])
        plsc.subcore_barrier()

        @pl.loop(0, DEG_NLOC // DEG_RING)
        def _(t):
            hs = []
            for bq in range(DEG_RING):
                hs.append(pltpu.async_copy(
                    ones_v, acc_sh.at[di_all.at[t * DEG_RING + bq]],
                    sems[bq], add=True))
            for bq in range(DEG_RING):
                hs[bq].wait()

        plsc.subcore_barrier()
        pltpu.sync_copy(acc_sh.at[pl.ds(s * ZROWS, ZROWS)], out_hbm.at[c, s])

    return k(dst2d, ones_src, zeros_src)


def _sc_aggregate(g4, src2d, dst2d, zeros_src):
    mesh = plsc.VectorSubcoreMesh(core_axis_name="c", subcore_axis_name="s")

    @functools.partial(
        pl.kernel,
        mesh=mesh,
        out_type=jax.ShapeDtypeStruct((NSLAB, NS, ZROWS, DS), jnp.float32),
        scratch_types=[
            pltpu.VMEM((CHUNK,), jnp.int32),
            pltpu.VMEM((CHUNK,), jnp.int32),
            pltpu.VMEM((CHUNK, DS), jnp.float32),
            pltpu.VMEM_SHARED((NP, DS), jnp.float32),
            pltpu.SemaphoreType.DMA,
        ],
    )
    def k(g_hbm, src_hbm, dst_hbm, z_hbm, out_hbm, si_v, di_v, rows_v, acc_sh, sem):
        c = lax.axis_index("c")
        s = lax.axis_index("s")
        for p in range(NSLAB // NC):
            slab = c * (NSLAB // NC) + p
            pltpu.sync_copy(z_hbm, acc_sh.at[pl.ds(s * ZROWS, ZROWS)])
            plsc.subcore_barrier()

            @pl.loop(0, NCHP // NS)
            def _(kk):
                lc = s + kk * NS

                @pl.when(lc < NCHUNKS)
                def _():
                    pltpu.sync_copy(src_hbm.at[lc], si_v)
                    pltpu.sync_copy(dst_hbm.at[lc], di_v)
                    pltpu.async_copy(g_hbm.at[slab].at[si_v], rows_v, sem).wait()
                    pltpu.sync_copy(rows_v, acc_sh.at[di_v], add=True)

            plsc.subcore_barrier()
            pltpu.sync_copy(acc_sh.at[pl.ds(s * ZROWS, ZROWS)], out_hbm.at[slab, s])
            plsc.subcore_barrier()

    return k(g4, src2d, dst2d, zeros_src)


def _tc_transform(x, W, degp):
    def body(x_ref, w_ref, da_ref, db_ref, g_ref, slh_ref):
        h = jnp.dot(x_ref[...], w_ref[...], preferred_element_type=jnp.float32)
        deg = 1.0 + da_ref[0, :, 0] + db_ref[0, :, 0]
        dis = lax.rsqrt(deg)[:, None]
        g = h * dis
        slh_ref[...] = g * dis
        for p in range(NSLAB):
            g_ref[p, :, :] = g[:, p * DS:(p + 1) * DS]

    return pl.pallas_call(
        body,
        grid=(N // BM,),
        in_specs=[
            pl.BlockSpec((BM, D), lambda i: (i, 0)),
            pl.BlockSpec((D, D), lambda i: (0, 0)),
            pl.BlockSpec((1, BM, DS), lambda i: (0, i, 0)),
            pl.BlockSpec((1, BM, DS), lambda i: (1, i, 0)),
        ],
        out_specs=[
            pl.BlockSpec((NSLAB, BM, DS), lambda i: (0, i, 0)),
            pl.BlockSpec((BM, D), lambda i: (i, 0)),
        ],
        out_shape=[
            jax.ShapeDtypeStruct((NSLAB, N, DS), jnp.float32),
            jax.ShapeDtypeStruct((N, D), jnp.float32),
        ],
    )(x, W, degp, degp)


def _tc_combine(acc4, slh, degp, b_row):
    def body(a_ref, slh_ref, da_ref, db_ref, b_ref, o_ref):
        deg = 1.0 + da_ref[0, :, 0] + db_ref[0, :, 0]
        dis = lax.rsqrt(deg)[:, None]
        acc = jnp.concatenate([a_ref[p] for p in range(NSLAB)], axis=1)
        o_ref[...] = acc * dis + slh_ref[...] + b_ref[...]

    return pl.pallas_call(
        body,
        grid=(N // BM,),
        in_specs=[
            pl.BlockSpec((NSLAB, BM, DS), lambda i: (0, i, 0)),
            pl.BlockSpec((BM, D), lambda i: (i, 0)),
            pl.BlockSpec((1, BM, DS), lambda i: (0, i, 0)),
            pl.BlockSpec((1, BM, DS), lambda i: (1, i, 0)),
            pl.BlockSpec((1, D), lambda i: (0, 0)),
        ],
        out_specs=pl.BlockSpec((BM, D), lambda i: (i, 0)),
        out_shape=jax.ShapeDtypeStruct((N, D), jnp.float32),
    )(acc4, slh, degp, degp, b_row)


def kernel(x, edge_index, W, b):
    src = edge_index[0].astype(jnp.int32)
    dst = edge_index[1].astype(jnp.int32)
    src2 = jnp.concatenate(
        [src, jnp.zeros((E_PAD - E,), jnp.int32)]).reshape(NCHP, CHUNK)
    pad_dst = N + (jnp.arange(E_PAD - E, dtype=jnp.int32) % 128)
    dst2 = jnp.concatenate([dst, pad_dst]).reshape(NCHP, CHUNK)
    ones_src = jnp.ones((CHUNK, DS), jnp.float32)
    zeros_rows = jnp.zeros((ZROWS, DS), jnp.float32)

    degp = _sc_degree(dst2, ones_src, zeros_rows).reshape(NC, N, DS)
    g4, slh = _tc_transform(x, W, degp)
    acc4 = _sc_aggregate(g4, src2, dst2, zeros_rows).reshape(NSLAB, N, DS)
    out = _tc_combine(acc4, slh, degp, b[None, :])
    return out
